# baseline (speedup 1.0000x reference)
import jax
import jax.numpy as jnp
from jax.experimental import pallas as pl
from jax.experimental.pallas import tpu as pltpu


def _fused_kernel(lbl_ref, x_any, w1_ref, w2_ref, feat_ref, lab_ref,
                  xbuf, sems):
    b = pl.program_id(0)
    nb = pl.num_programs(0)
    slot = jax.lax.rem(b, 2)
    c, h, w = x_any.shape[1:]

    @pl.when(b == 0)
    def _start_first():
        pltpu.make_async_copy(x_any.at[0], xbuf.at[0], sems.at[0]).start()

    @pl.when(b + 1 < nb)
    def _prefetch_next():
        nxt = jax.lax.rem(b + 1, 2)
        pltpu.make_async_copy(x_any.at[b + 1], xbuf.at[nxt], sems.at[nxt]).start()

    pltpu.make_async_copy(xbuf.at[slot], xbuf.at[slot], sems.at[slot]).wait()

    xb = xbuf[slot].astype(jnp.bfloat16).reshape(c, h * w)
    hid = jnp.dot(w1_ref[...], xb, preferred_element_type=jnp.float32)
    hb = jnp.maximum(hid, 0.0).astype(jnp.bfloat16)
    feat_ref[0] = jnp.dot(w2_ref[...], hb, preferred_element_type=jnp.float32)
    lab_ref[...] = jnp.full(lab_ref.shape, lbl_ref[b], dtype=jnp.int32)


def kernel(x, w1, w2, need_backprop):
    B, C, H, W = x.shape
    hidden = w1.shape[0]
    out_c = w2.shape[0]
    HW = H * W

    lbl = need_backprop.astype(jnp.float32).astype(jnp.int32)
    w1b = w1.astype(jnp.bfloat16)
    w2b = w2.astype(jnp.bfloat16)

    feat, lab = pl.pallas_call(
        _fused_kernel,
        out_shape=(
            jax.ShapeDtypeStruct((B, out_c, HW), jnp.float32),
            jax.ShapeDtypeStruct((B, 1, HW), jnp.int32),
        ),
        grid_spec=pltpu.PrefetchScalarGridSpec(
            num_scalar_prefetch=1,
            grid=(B,),
            in_specs=[
                pl.BlockSpec(memory_space=pl.ANY),
                pl.BlockSpec((hidden, C), lambda b, lbl: (0, 0)),
                pl.BlockSpec((out_c, hidden), lambda b, lbl: (0, 0)),
            ],
            out_specs=(
                pl.BlockSpec((1, out_c, HW), lambda b, lbl: (b, 0, 0)),
                pl.BlockSpec((1, 1, HW), lambda b, lbl: (b, 0, 0)),
            ),
            scratch_shapes=[
                pltpu.VMEM((2, C, H, W), jnp.float32),
                pltpu.SemaphoreType.DMA((2,)),
            ],
        ),
        compiler_params=pltpu.CompilerParams(
            dimension_semantics=("arbitrary",)),
    )(lbl, x, w1b, w2b)

    return feat.reshape(B, out_c, H, W), lab.reshape(B, H, W)


# speedup vs baseline: 1.6441x; 1.6441x over previous
import jax
import jax.numpy as jnp
from jax.experimental import pallas as pl
from jax.experimental.pallas import tpu as pltpu

_NCHUNK = 4


def _fused_kernel(lbl_ref, x_any, w1_ref, w2_ref, feat_ref, lab_ref,
                  xbuf, sems):
    b = pl.program_id(0)
    nb = pl.num_programs(0)
    slot = jax.lax.rem(b, 2)
    hw = x_any.shape[2]
    ck = hw // _NCHUNK

    @pl.when(b == 0)
    def _start_first():
        pltpu.make_async_copy(x_any.at[0], xbuf.at[0], sems.at[0]).start()

    @pl.when(b + 1 < nb)
    def _prefetch_next():
        nxt = jax.lax.rem(b + 1, 2)
        pltpu.make_async_copy(x_any.at[b + 1], xbuf.at[nxt], sems.at[nxt]).start()

    pltpu.make_async_copy(xbuf.at[slot], xbuf.at[slot], sems.at[slot]).wait()

    for i in range(_NCHUNK):
        xi = xbuf[slot, :, i * ck:(i + 1) * ck].astype(jnp.bfloat16)
        hi = jnp.dot(w1_ref[...], xi, preferred_element_type=jnp.float32)
        hb = jnp.maximum(hi.astype(jnp.bfloat16), 0.0)
        feat_ref[0, :, i * ck:(i + 1) * ck] = jnp.dot(
            w2_ref[...], hb, preferred_element_type=jnp.float32)
    lab_ref[...] = jnp.full(lab_ref.shape, lbl_ref[b], dtype=jnp.int32)


def kernel(x, w1, w2, need_backprop):
    B, C, H, W = x.shape
    hidden = w1.shape[0]
    out_c = w2.shape[0]
    HW = H * W

    x_r = x.reshape(B, C, HW)

    lbl = need_backprop.astype(jnp.float32).astype(jnp.int32)
    w1b = w1.astype(jnp.bfloat16)
    w2b = w2.astype(jnp.bfloat16)

    feat, lab = pl.pallas_call(
        _fused_kernel,
        out_shape=(
            jax.ShapeDtypeStruct((B, out_c, HW), jnp.float32),
            jax.ShapeDtypeStruct((B, 1, HW), jnp.int32),
        ),
        grid_spec=pltpu.PrefetchScalarGridSpec(
            num_scalar_prefetch=1,
            grid=(B,),
            in_specs=[
                pl.BlockSpec(memory_space=pl.ANY),
                pl.BlockSpec((hidden, C), lambda b, lbl: (0, 0)),
                pl.BlockSpec((out_c, hidden), lambda b, lbl: (0, 0)),
            ],
            out_specs=(
                pl.BlockSpec((1, out_c, HW), lambda b, lbl: (b, 0, 0)),
                pl.BlockSpec((1, 1, HW), lambda b, lbl: (b, 0, 0)),
            ),
            scratch_shapes=[
                pltpu.VMEM((2, C, HW), jnp.float32),
                pltpu.SemaphoreType.DMA((2,)),
            ],
        ),
        compiler_params=pltpu.CompilerParams(
            dimension_semantics=("arbitrary",)),
    )(lbl, x_r, w1b, w2b)

    return feat.reshape(B, out_c, H, W), lab.reshape(B, H, W)


# speedup vs baseline: 1.6800x; 1.0218x over previous
import jax
import jax.numpy as jnp
from jax.experimental import pallas as pl
from jax.experimental.pallas import tpu as pltpu

_NCHUNK = 1


def _fused_kernel(lbl_ref, x_any, w1_ref, w2_ref, feat_ref, lab_ref,
                  xbuf, sems):
    b = pl.program_id(0)
    nb = pl.num_programs(0)
    slot = jax.lax.rem(b, 2)
    hw = x_any.shape[2]
    ck = hw // _NCHUNK

    @pl.when(b == 0)
    def _start_first():
        pltpu.make_async_copy(x_any.at[0], xbuf.at[0], sems.at[0]).start()

    @pl.when(b + 1 < nb)
    def _prefetch_next():
        nxt = jax.lax.rem(b + 1, 2)
        pltpu.make_async_copy(x_any.at[b + 1], xbuf.at[nxt], sems.at[nxt]).start()

    pltpu.make_async_copy(xbuf.at[slot], xbuf.at[slot], sems.at[slot]).wait()

    for i in range(_NCHUNK):
        xi = xbuf[slot, :, i * ck:(i + 1) * ck].astype(jnp.bfloat16)
        hi = jnp.dot(w1_ref[...], xi, preferred_element_type=jnp.float32)
        hb = jnp.maximum(hi.astype(jnp.bfloat16), 0.0)
        feat_ref[0, :, i * ck:(i + 1) * ck] = jnp.dot(
            w2_ref[...], hb, preferred_element_type=jnp.float32)
    lab_ref[...] = jnp.full(lab_ref.shape, lbl_ref[b], dtype=jnp.int32)


def kernel(x, w1, w2, need_backprop):
    B, C, H, W = x.shape
    hidden = w1.shape[0]
    out_c = w2.shape[0]
    HW = H * W

    x_r = x.reshape(B, C, HW)

    lbl = need_backprop.astype(jnp.float32).astype(jnp.int32)
    w1b = w1.astype(jnp.bfloat16)
    w2b = w2.astype(jnp.bfloat16)

    feat, lab = pl.pallas_call(
        _fused_kernel,
        out_shape=(
            jax.ShapeDtypeStruct((B, out_c, HW), jnp.float32),
            jax.ShapeDtypeStruct((B, 1, HW), jnp.int32),
        ),
        grid_spec=pltpu.PrefetchScalarGridSpec(
            num_scalar_prefetch=1,
            grid=(B,),
            in_specs=[
                pl.BlockSpec(memory_space=pl.ANY),
                pl.BlockSpec((hidden, C), lambda b, lbl: (0, 0)),
                pl.BlockSpec((out_c, hidden), lambda b, lbl: (0, 0)),
            ],
            out_specs=(
                pl.BlockSpec((1, out_c, HW), lambda b, lbl: (b, 0, 0)),
                pl.BlockSpec((1, 1, HW), lambda b, lbl: (b, 0, 0)),
            ),
            scratch_shapes=[
                pltpu.VMEM((2, C, HW), jnp.float32),
                pltpu.SemaphoreType.DMA((2,)),
            ],
        ),
        compiler_params=pltpu.CompilerParams(
            dimension_semantics=("arbitrary",)),
    )(lbl, x_r, w1b, w2b)

    return feat.reshape(B, out_c, H, W), lab.reshape(B, H, W)


# speedup vs baseline: 3.7628x; 2.2398x over previous
import jax
import jax.numpy as jnp
from jax.experimental import pallas as pl
from jax.experimental.pallas import tpu as pltpu


def _fused_kernel(lbl_ref, x_any, w1_ref, w2_ref, feat_ref, lab_ref,
                  xbuf, sems):
    b = pl.program_id(0)
    nb = pl.num_programs(0)
    slot = jax.lax.rem(b, 2)

    @pl.when(b == 0)
    def _start_first():
        pltpu.make_async_copy(x_any.at[0], xbuf.at[0], sems.at[0]).start()

    @pl.when(b + 1 < nb)
    def _prefetch_next():
        nxt = jax.lax.rem(b + 1, 2)
        pltpu.make_async_copy(x_any.at[b + 1], xbuf.at[nxt], sems.at[nxt]).start()

    pltpu.make_async_copy(xbuf.at[slot], xbuf.at[slot], sems.at[slot]).wait()

    xb = xbuf[slot].astype(jnp.bfloat16)
    hid = jax.lax.dot_general(
        w1_ref[...], xb, (((1,), (1,)), ((), ())),
        preferred_element_type=jnp.float32)
    hb = jnp.maximum(hid.astype(jnp.bfloat16), 0.0)
    feat_ref[0] = jnp.dot(w2_ref[...], hb, preferred_element_type=jnp.float32)
    lab_ref[...] = jnp.full(lab_ref.shape, lbl_ref[b], dtype=jnp.int32)


def kernel(x, w1, w2, need_backprop):
    B, C, H, W = x.shape
    hidden = w1.shape[0]
    out_c = w2.shape[0]
    HW = H * W

    x_t = jnp.transpose(x, (0, 2, 3, 1)).reshape(B, HW, C)

    lbl = need_backprop.astype(jnp.float32).astype(jnp.int32)
    w1b = w1.astype(jnp.bfloat16)
    w2b = w2.astype(jnp.bfloat16)

    feat, lab = pl.pallas_call(
        _fused_kernel,
        out_shape=(
            jax.ShapeDtypeStruct((B, out_c, HW), jnp.float32),
            jax.ShapeDtypeStruct((B, 1, HW), jnp.int32),
        ),
        grid_spec=pltpu.PrefetchScalarGridSpec(
            num_scalar_prefetch=1,
            grid=(B,),
            in_specs=[
                pl.BlockSpec(memory_space=pl.ANY),
                pl.BlockSpec((hidden, C), lambda b, lbl: (0, 0)),
                pl.BlockSpec((out_c, hidden), lambda b, lbl: (0, 0)),
            ],
            out_specs=(
                pl.BlockSpec((1, out_c, HW), lambda b, lbl: (b, 0, 0)),
                pl.BlockSpec((1, 1, HW), lambda b, lbl: (b, 0, 0)),
            ),
            scratch_shapes=[
                pltpu.VMEM((2, HW, C), jnp.float32),
                pltpu.SemaphoreType.DMA((2,)),
            ],
        ),
        compiler_params=pltpu.CompilerParams(
            dimension_semantics=("arbitrary",)),
    )(lbl, x_t, w1b, w2b)

    return feat.reshape(B, out_c, H, W), lab.reshape(B, H, W)


# speedup vs baseline: 4.5085x; 1.1982x over previous
import jax
import jax.numpy as jnp
from jax.experimental import pallas as pl
from jax.experimental.pallas import tpu as pltpu


def _fused_kernel(nbp_ref, x_any, w1_ref, w2_ref, feat_ref, lab_ref,
                  xbuf, sems):
    b = pl.program_id(0)
    nb = pl.num_programs(0)
    slot = jax.lax.rem(b, 2)
    h_dim, w_dim = feat_ref.shape[2], feat_ref.shape[3]

    @pl.when(b == 0)
    def _start_first():
        pltpu.make_async_copy(x_any.at[0], xbuf.at[0], sems.at[0]).start()

    @pl.when(b + 1 < nb)
    def _prefetch_next():
        nxt = jax.lax.rem(b + 1, 2)
        pltpu.make_async_copy(x_any.at[b + 1], xbuf.at[nxt], sems.at[nxt]).start()

    pltpu.make_async_copy(xbuf.at[slot], xbuf.at[slot], sems.at[slot]).wait()

    xb = xbuf[slot].astype(jnp.bfloat16)
    hid = jax.lax.dot_general(
        w1_ref[...].astype(jnp.bfloat16), xb, (((1,), (1,)), ((), ())),
        preferred_element_type=jnp.float32)
    hb = jnp.maximum(hid.astype(jnp.bfloat16), 0.0)
    out = jnp.dot(w2_ref[...].astype(jnp.bfloat16), hb,
                  preferred_element_type=jnp.float32)
    feat_ref[0] = out.reshape(out.shape[0], h_dim, w_dim)
    lab_ref[...] = jnp.full(lab_ref.shape, nbp_ref[b].astype(jnp.int32),
                            dtype=jnp.int32)


def kernel(x, w1, w2, need_backprop):
    B, C, H, W = x.shape
    hidden = w1.shape[0]
    out_c = w2.shape[0]
    HW = H * W

    x_t = jnp.transpose(x, (0, 2, 3, 1)).reshape(B, HW, C)

    feat, lab = pl.pallas_call(
        _fused_kernel,
        out_shape=(
            jax.ShapeDtypeStruct((B, out_c, H, W), jnp.float32),
            jax.ShapeDtypeStruct((B, H, W), jnp.int32),
        ),
        grid_spec=pltpu.PrefetchScalarGridSpec(
            num_scalar_prefetch=1,
            grid=(B,),
            in_specs=[
                pl.BlockSpec(memory_space=pl.ANY),
                pl.BlockSpec((hidden, C), lambda b, nbp: (0, 0)),
                pl.BlockSpec((out_c, hidden), lambda b, nbp: (0, 0)),
            ],
            out_specs=(
                pl.BlockSpec((1, out_c, H, W), lambda b, nbp: (b, 0, 0, 0)),
                pl.BlockSpec((1, H, W), lambda b, nbp: (b, 0, 0)),
            ),
            scratch_shapes=[
                pltpu.VMEM((2, HW, C), jnp.float32),
                pltpu.SemaphoreType.DMA((2,)),
            ],
        ),
        compiler_params=pltpu.CompilerParams(
            dimension_semantics=("arbitrary",)),
    )(need_backprop, x_t, w1, w2)

    return feat, lab


# speedup vs baseline: 4.8689x; 1.0799x over previous
import jax
import jax.numpy as jnp
from jax.experimental import pallas as pl
from jax.experimental.pallas import tpu as pltpu

_PB = 2


def _fused_kernel(nbp_ref, x_any, w1_ref, w2_ref, feat_ref, lab_ref,
                  xbuf, sems):
    g = pl.program_id(0)
    ng = pl.num_programs(0)
    slot = jax.lax.rem(g, 2)
    pb, oc, h_dim, w_dim = feat_ref.shape
    hw = h_dim * w_dim

    @pl.when(g == 0)
    def _start_first():
        pltpu.make_async_copy(x_any.at[0], xbuf.at[0], sems.at[0]).start()

    @pl.when(g + 1 < ng)
    def _prefetch_next():
        nxt = jax.lax.rem(g + 1, 2)
        pltpu.make_async_copy(x_any.at[g + 1], xbuf.at[nxt], sems.at[nxt]).start()

    pltpu.make_async_copy(xbuf.at[slot], xbuf.at[slot], sems.at[slot]).wait()

    xb = xbuf[slot].astype(jnp.bfloat16)
    hid = jax.lax.dot_general(
        w1_ref[...].astype(jnp.bfloat16), xb, (((1,), (1,)), ((), ())),
        preferred_element_type=jnp.float32)
    hb = jnp.maximum(hid.astype(jnp.bfloat16), 0.0)
    out = jnp.dot(w2_ref[...].astype(jnp.bfloat16), hb,
                  preferred_element_type=jnp.float32)
    for p in range(pb):
        feat_ref[p] = out[:, p * hw:(p + 1) * hw].reshape(oc, h_dim, w_dim)
        lab_ref[p] = jnp.full(
            (h_dim, w_dim), nbp_ref[g * pb + p].astype(jnp.int32),
            dtype=jnp.int32)


def kernel(x, w1, w2, need_backprop):
    B, C, H, W = x.shape
    hidden = w1.shape[0]
    out_c = w2.shape[0]
    HW = H * W
    pb = _PB if B % _PB == 0 else 1

    x_t = jnp.transpose(x, (0, 2, 3, 1)).reshape(B // pb, pb * HW, C)

    feat, lab = pl.pallas_call(
        _fused_kernel,
        out_shape=(
            jax.ShapeDtypeStruct((B, out_c, H, W), jnp.float32),
            jax.ShapeDtypeStruct((B, H, W), jnp.int32),
        ),
        grid_spec=pltpu.PrefetchScalarGridSpec(
            num_scalar_prefetch=1,
            grid=(B // pb,),
            in_specs=[
                pl.BlockSpec(memory_space=pl.ANY),
                pl.BlockSpec((hidden, C), lambda g, nbp: (0, 0)),
                pl.BlockSpec((out_c, hidden), lambda g, nbp: (0, 0)),
            ],
            out_specs=(
                pl.BlockSpec((pb, out_c, H, W), lambda g, nbp: (g, 0, 0, 0)),
                pl.BlockSpec((pb, H, W), lambda g, nbp: (g, 0, 0)),
            ),
            scratch_shapes=[
                pltpu.VMEM((2, pb * HW, C), jnp.float32),
                pltpu.SemaphoreType.DMA((2,)),
            ],
        ),
        compiler_params=pltpu.CompilerParams(
            dimension_semantics=("arbitrary",)),
    )(need_backprop, x_t, w1, w2)

    return feat, lab


# speedup vs baseline: 4.8749x; 1.0012x over previous
import jax
import jax.numpy as jnp
from jax.experimental import pallas as pl
from jax.experimental.pallas import tpu as pltpu

_PB = 2


def _fused_kernel(nbp_ref, x_any, w1_ref, w2_ref, feat_ref, lab_ref,
                  xbuf, sems):
    g = pl.program_id(0)
    ng = pl.num_programs(0)
    slot = jax.lax.rem(g, 3)
    pb, oc, h_dim, w_dim = feat_ref.shape
    hw = h_dim * w_dim

    @pl.when(g == 0)
    def _start_first():
        pltpu.make_async_copy(x_any.at[0], xbuf.at[0], sems.at[0]).start()
        if ng > 1:
            pltpu.make_async_copy(x_any.at[1], xbuf.at[1], sems.at[1]).start()

    @pl.when(g + 2 < ng)
    def _prefetch_next():
        nxt = jax.lax.rem(g + 2, 3)
        pltpu.make_async_copy(x_any.at[g + 2], xbuf.at[nxt], sems.at[nxt]).start()

    pltpu.make_async_copy(xbuf.at[slot], xbuf.at[slot], sems.at[slot]).wait()

    xb = xbuf[slot].astype(jnp.bfloat16)
    hid = jax.lax.dot_general(
        w1_ref[...].astype(jnp.bfloat16), xb, (((1,), (1,)), ((), ())),
        preferred_element_type=jnp.float32)
    hb = jnp.maximum(hid.astype(jnp.bfloat16), 0.0)
    out = jnp.dot(w2_ref[...].astype(jnp.bfloat16), hb,
                  preferred_element_type=jnp.float32)
    for p in range(pb):
        feat_ref[p] = out[:, p * hw:(p + 1) * hw].reshape(oc, h_dim, w_dim)
        lab_ref[p] = jnp.full(
            (h_dim, w_dim), nbp_ref[g * pb + p].astype(jnp.int32),
            dtype=jnp.int32)


def kernel(x, w1, w2, need_backprop):
    B, C, H, W = x.shape
    hidden = w1.shape[0]
    out_c = w2.shape[0]
    HW = H * W
    pb = _PB if B % _PB == 0 else 1

    x_t = jnp.transpose(x, (0, 2, 3, 1)).reshape(B // pb, pb * HW, C)

    feat, lab = pl.pallas_call(
        _fused_kernel,
        out_shape=(
            jax.ShapeDtypeStruct((B, out_c, H, W), jnp.float32),
            jax.ShapeDtypeStruct((B, H, W), jnp.int32),
        ),
        grid_spec=pltpu.PrefetchScalarGridSpec(
            num_scalar_prefetch=1,
            grid=(B // pb,),
            in_specs=[
                pl.BlockSpec(memory_space=pl.ANY),
                pl.BlockSpec((hidden, C), lambda g, nbp: (0, 0)),
                pl.BlockSpec((out_c, hidden), lambda g, nbp: (0, 0)),
            ],
            out_specs=(
                pl.BlockSpec((pb, out_c, H, W), lambda g, nbp: (g, 0, 0, 0)),
                pl.BlockSpec((pb, H, W), lambda g, nbp: (g, 0, 0)),
            ),
            scratch_shapes=[
                pltpu.VMEM((3, pb * HW, C), jnp.float32),
                pltpu.SemaphoreType.DMA((3,)),
            ],
        ),
        compiler_params=pltpu.CompilerParams(
            dimension_semantics=("arbitrary",)),
    )(need_backprop, x_t, w1, w2)

    return feat, lab


# speedup vs baseline: 4.8809x; 1.0012x over previous
import jax
import jax.numpy as jnp
from jax.experimental import pallas as pl
from jax.experimental.pallas import tpu as pltpu

_PB = 2


def _fused_kernel(nbp_ref, x_any, w1_ref, w2_ref, feat_ref, lab_ref,
                  xbuf, sems):
    g = pl.program_id(0)
    ng = pl.num_programs(0)
    slot = jax.lax.rem(g, 2)
    pb, oc, h_dim, w_dim = feat_ref.shape
    hw = h_dim * w_dim

    @pl.when(g == 0)
    def _start_first():
        pltpu.make_async_copy(x_any.at[0], xbuf.at[0], sems.at[0]).start()

    @pl.when(g + 1 < ng)
    def _prefetch_next():
        nxt = jax.lax.rem(g + 1, 2)
        pltpu.make_async_copy(x_any.at[g + 1], xbuf.at[nxt], sems.at[nxt]).start()

    pltpu.make_async_copy(xbuf.at[slot], xbuf.at[slot], sems.at[slot]).wait()

    xb = xbuf[slot].astype(jnp.bfloat16)
    hid = jax.lax.dot_general(
        w1_ref[...].astype(jnp.bfloat16), xb, (((1,), (1,)), ((), ())),
        preferred_element_type=jnp.float32)
    hb = jnp.maximum(hid.astype(jnp.bfloat16), 0.0)
    out = jnp.dot(w2_ref[...].astype(jnp.bfloat16), hb,
                  preferred_element_type=jnp.float32)
    for p in range(pb):
        feat_ref[p] = out[:, p * hw:(p + 1) * hw].reshape(oc, h_dim, w_dim)
        lab_ref[p] = jnp.full(
            (h_dim, w_dim), nbp_ref[g * pb + p].astype(jnp.int32),
            dtype=jnp.int32)


def kernel(x, w1, w2, need_backprop):
    B, C, H, W = x.shape
    hidden = w1.shape[0]
    out_c = w2.shape[0]
    HW = H * W
    pb = _PB if B % _PB == 0 else 1

    x_t = jnp.transpose(x, (0, 2, 3, 1)).reshape(B // pb, pb * HW, C)

    feat, lab = pl.pallas_call(
        _fused_kernel,
        out_shape=(
            jax.ShapeDtypeStruct((B, out_c, H, W), jnp.float32),
            jax.ShapeDtypeStruct((B, H, W), jnp.int32),
        ),
        grid_spec=pltpu.PrefetchScalarGridSpec(
            num_scalar_prefetch=1,
            grid=(B // pb,),
            in_specs=[
                pl.BlockSpec(memory_space=pl.ANY),
                pl.BlockSpec((hidden, C), lambda g, nbp: (0, 0)),
                pl.BlockSpec((out_c, hidden), lambda g, nbp: (0, 0)),
            ],
            out_specs=(
                pl.BlockSpec((pb, out_c, H, W), lambda g, nbp: (g, 0, 0, 0)),
                pl.BlockSpec((pb, H, W), lambda g, nbp: (g, 0, 0)),
            ),
            scratch_shapes=[
                pltpu.VMEM((2, pb * HW, C), jnp.float32),
                pltpu.SemaphoreType.DMA((2,)),
            ],
        ),
        compiler_params=pltpu.CompilerParams(
            dimension_semantics=("arbitrary",)),
    )(need_backprop, x_t, w1, w2)

    return feat, lab
